# baseline (speedup 1.0000x reference)
import functools

import jax
import jax.numpy as jnp
from jax import lax
from jax.experimental import pallas as pl
from jax.experimental.pallas import tpu as pltpu
from jax.experimental.pallas import tpu_sc as plsc

SL = 8192
BS = 4
HIDDEN = 1024
EXPERTS = 8
TOPK = 2
N = SL * BS

_TOL = 1e-4
_EPS = 1e-8

NC = 2
NS = 16
L = 16
NW = NC * NS
TPW = N // NW
TB = 16
TG = 4
HC = HIDDEN // L


def _round_bf16(v):
    u = lax.bitcast_convert_type(v, jnp.uint32)
    lsb = lax.shift_right_logical(u, jnp.uint32(16)) & jnp.uint32(1)
    u = (u + jnp.uint32(0x7FFF) + lsb) & jnp.uint32(0xFFFF0000)
    return lax.bitcast_convert_type(u, jnp.float32)


def _sc_cost_kernel(x_hbm, w_hbm, out_hbm, w_v, x_v, stage_v, o_v):
    cid = lax.axis_index("c")
    sid = lax.axis_index("s")
    wid = sid * NC + cid
    base = wid * TPW
    pltpu.sync_copy(w_hbm, w_v)
    for e in range(EXPERTS):

        def w_round(h, carry, e=e):
            w_v[e, pl.ds(h * L, L)] = _round_bf16(w_v[e, pl.ds(h * L, L)])
            return carry

        lax.fori_loop(0, HC, w_round, 0)
    lane = lax.iota(jnp.int32, L)

    def blk_body(blk, carry):
        pltpu.sync_copy(x_hbm.at[pl.ds(base + blk * TB, TB)], x_v)
        for g in range(TB // TG):

            def h_body(h, accs):
                xcs = [
                    _round_bf16(x_v[g * TG + t, pl.ds(h * L, L)])
                    for t in range(TG)
                ]
                wcs = [w_v[e, pl.ds(h * L, L)] for e in range(EXPERTS)]
                return tuple(
                    accs[t * EXPERTS + e] + xcs[t] * wcs[e]
                    for t in range(TG)
                    for e in range(EXPERTS)
                )

            zero = jnp.zeros((L,), jnp.float32)
            accs = lax.fori_loop(0, HC, h_body, (zero,) * (TG * EXPERTS))
            for t in range(TG):
                r = zero
                for e in range(EXPERTS):
                    r = jnp.where(lane == e, jnp.sum(accs[t * EXPERTS + e]), r)
                stage_v[g * TG + t, :] = r
        for e in range(EXPERTS):
            ge = plsc.load_gather(stage_v, [lane, jnp.full((L,), e, jnp.int32)])
            o_v[e, pl.ds(blk * TB, TB)] = jnp.exp(ge)
        return carry

    lax.fori_loop(0, TPW // TB, blk_body, 0)
    for e in range(EXPERTS):
        pltpu.sync_copy(o_v.at[e], out_hbm.at[e, pl.ds(base, TPW)])


_sc_cost = functools.partial(
    pl.kernel,
    out_type=jax.ShapeDtypeStruct((EXPERTS, N), jnp.float32),
    mesh=plsc.VectorSubcoreMesh(
        core_axis_name="c", subcore_axis_name="s", num_cores=NC, num_subcores=NS
    ),
    scratch_types=[
        pltpu.VMEM((EXPERTS, HIDDEN), jnp.float32),
        pltpu.VMEM((TB, HIDDEN), jnp.float32),
        pltpu.VMEM((TB, L), jnp.float32),
        pltpu.VMEM((EXPERTS, TPW), jnp.float32),
    ],
    compiler_params=pltpu.CompilerParams(needs_layout_passes=False),
)(_sc_cost_kernel)


def _finish_kernel(cost_ref, scores_ref, idx_ref):
    cost = cost_ref[...]

    def cond_fn(carry):
        return carry[1] > _TOL

    def body_fn(carry):
        d1, _ = carry
        rowsum = jnp.sum(d1 * cost, axis=0, keepdims=True)
        d0 = (1.0 / N) / (rowsum + _EPS)
        colsum = jnp.sum(d0 * cost, axis=1, keepdims=True)
        d1n = (1.0 / EXPERTS) / (colsum + _EPS)
        err = jnp.mean(jnp.abs(d1 - d1n))
        return (d1n, err)

    d1_init = jnp.ones((EXPERTS, 1), jnp.float32)
    d1, _ = lax.while_loop(cond_fn, body_fn, (d1_init, jnp.float32(1e9)))

    s = d1 * cost
    eidx = lax.broadcasted_iota(jnp.int32, (EXPERTS, N), 0)
    m1 = jnp.max(s, axis=0, keepdims=True)
    i1 = jnp.min(jnp.where(s == m1, eidx, EXPERTS), axis=0, keepdims=True)
    masked = jnp.where(eidx == i1, -jnp.inf, s)
    m2 = jnp.max(masked, axis=0, keepdims=True)
    i2 = jnp.min(jnp.where(masked == m2, eidx, EXPERTS), axis=0, keepdims=True)

    denom = jnp.sum(cost, axis=0, keepdims=True)
    c1 = jnp.sum(jnp.where(eidx == i1, cost, 0.0), axis=0, keepdims=True)
    c2 = jnp.sum(jnp.where(eidx == i2, cost, 0.0), axis=0, keepdims=True)
    scores_ref[...] = jnp.concatenate([c1 / denom, c2 / denom], axis=0)
    idx_ref[...] = jnp.concatenate([i1, i2], axis=0)


def kernel(x, W):
    x2d = x.reshape(-1, HIDDEN)
    cost = _sc_cost(x2d, W)
    scores_t, idx_t = pl.pallas_call(
        _finish_kernel,
        out_shape=[
            jax.ShapeDtypeStruct((TOPK, N), jnp.float32),
            jax.ShapeDtypeStruct((TOPK, N), jnp.int32),
        ],
    )(cost)
    return (scores_t.T, idx_t.T)


# speedup vs baseline: 1.9977x; 1.9977x over previous
import functools

import jax
import jax.numpy as jnp
from jax import lax
from jax.experimental import pallas as pl
from jax.experimental.pallas import tpu as pltpu
from jax.experimental.pallas import tpu_sc as plsc

SL = 8192
BS = 4
HIDDEN = 1024
EXPERTS = 8
TOPK = 2
N = SL * BS

_TOL = 1e-4
_EPS = 1e-8

N_SC = 8192
N_TC = N - N_SC

NC = 2
NS = 16
L = 16
NW = NC * NS
TPW = N_SC // NW
TB = 16
TG = 4
HC = HIDDEN // L

ROWS = 2048
NT = N_TC // ROWS


def _round_bf16(v):
    u = lax.bitcast_convert_type(v, jnp.uint32)
    lsb = lax.shift_right_logical(u, jnp.uint32(16)) & jnp.uint32(1)
    u = (u + jnp.uint32(0x7FFF) + lsb) & jnp.uint32(0xFFFF0000)
    return lax.bitcast_convert_type(u, jnp.float32)


def _sc_cost_kernel(x_hbm, w_hbm, out_hbm, w_v, x_v, stage_v, o_v):
    cid = lax.axis_index("c")
    sid = lax.axis_index("s")
    wid = sid * NC + cid
    base = wid * TPW
    pltpu.sync_copy(w_hbm, w_v)
    for e in range(EXPERTS):

        def w_round(h, carry, e=e):
            w_v[e, pl.ds(h * L, L)] = _round_bf16(w_v[e, pl.ds(h * L, L)])
            return carry

        lax.fori_loop(0, HC, w_round, 0)
    lane = lax.iota(jnp.int32, L)

    def blk_body(blk, carry):
        pltpu.sync_copy(x_hbm.at[pl.ds(base + blk * TB, TB)], x_v)
        for g in range(TB // TG):

            def h_body(h, accs):
                xcs = [
                    _round_bf16(x_v[g * TG + t, pl.ds(h * L, L)])
                    for t in range(TG)
                ]
                wcs = [w_v[e, pl.ds(h * L, L)] for e in range(EXPERTS)]
                return tuple(
                    accs[t * EXPERTS + e] + xcs[t] * wcs[e]
                    for t in range(TG)
                    for e in range(EXPERTS)
                )

            zero = jnp.zeros((L,), jnp.float32)
            accs = lax.fori_loop(0, HC, h_body, (zero,) * (TG * EXPERTS))
            for t in range(TG):
                r = zero
                for e in range(EXPERTS):
                    r = jnp.where(lane == e, jnp.sum(accs[t * EXPERTS + e]), r)
                stage_v[g * TG + t, :] = r
        for e in range(EXPERTS):
            ge = plsc.load_gather(stage_v, [lane, jnp.full((L,), e, jnp.int32)])
            o_v[e, pl.ds(blk * TB, TB)] = jnp.exp(ge)
        return carry

    lax.fori_loop(0, TPW // TB, blk_body, 0)
    for e in range(EXPERTS):
        pltpu.sync_copy(o_v.at[e], out_hbm.at[e, pl.ds(base, TPW)])


_sc_cost = functools.partial(
    pl.kernel,
    out_type=jax.ShapeDtypeStruct((EXPERTS, N_SC), jnp.float32),
    mesh=plsc.VectorSubcoreMesh(
        core_axis_name="c", subcore_axis_name="s", num_cores=NC, num_subcores=NS
    ),
    scratch_types=[
        pltpu.VMEM((EXPERTS, HIDDEN), jnp.float32),
        pltpu.VMEM((TB, HIDDEN), jnp.float32),
        pltpu.VMEM((TB, L), jnp.float32),
        pltpu.VMEM((EXPERTS, TPW), jnp.float32),
    ],
    compiler_params=pltpu.CompilerParams(needs_layout_passes=False),
)(_sc_cost_kernel)


def _tc_cost_kernel(x_ref, w_ref, out_ref):
    x = x_ref[...]
    w = w_ref[...]
    logits_t = jax.lax.dot_general(
        w, x, (((1,), (1,)), ((), ())), preferred_element_type=jnp.float32
    )
    out_ref[...] = jnp.exp(logits_t)


def _finish_kernel(cost_a_ref, cost_b_ref, scores_ref, idx_ref):
    cost = jnp.concatenate([cost_a_ref[...], cost_b_ref[...]], axis=1)

    def cond_fn(carry):
        return carry[1] > _TOL

    def body_fn(carry):
        d1, _ = carry
        rowsum = jnp.sum(d1 * cost, axis=0, keepdims=True)
        d0 = (1.0 / N) / (rowsum + _EPS)
        colsum = jnp.sum(d0 * cost, axis=1, keepdims=True)
        d1n = (1.0 / EXPERTS) / (colsum + _EPS)
        err = jnp.mean(jnp.abs(d1 - d1n))
        return (d1n, err)

    d1_init = jnp.ones((EXPERTS, 1), jnp.float32)
    d1, _ = lax.while_loop(cond_fn, body_fn, (d1_init, jnp.float32(1e9)))

    s = d1 * cost
    eidx = lax.broadcasted_iota(jnp.int32, (EXPERTS, N), 0)
    m1 = jnp.max(s, axis=0, keepdims=True)
    i1 = jnp.min(jnp.where(s == m1, eidx, EXPERTS), axis=0, keepdims=True)
    masked = jnp.where(eidx == i1, -jnp.inf, s)
    m2 = jnp.max(masked, axis=0, keepdims=True)
    i2 = jnp.min(jnp.where(masked == m2, eidx, EXPERTS), axis=0, keepdims=True)

    denom = jnp.sum(cost, axis=0, keepdims=True)
    c1 = jnp.sum(jnp.where(eidx == i1, cost, 0.0), axis=0, keepdims=True)
    c2 = jnp.sum(jnp.where(eidx == i2, cost, 0.0), axis=0, keepdims=True)
    scores_ref[...] = jnp.concatenate([c1 / denom, c2 / denom], axis=0)
    idx_ref[...] = jnp.concatenate([i1, i2], axis=0)


def kernel(x, W):
    x2d = x.reshape(-1, HIDDEN)
    cost_sc = _sc_cost(x2d, W)
    cost_tc = pl.pallas_call(
        _tc_cost_kernel,
        grid=(NT,),
        in_specs=[
            pl.BlockSpec((ROWS, HIDDEN), lambda i: (i + N_SC // ROWS, 0)),
            pl.BlockSpec((EXPERTS, HIDDEN), lambda i: (0, 0)),
        ],
        out_specs=pl.BlockSpec((EXPERTS, ROWS), lambda i: (0, i)),
        out_shape=jax.ShapeDtypeStruct((EXPERTS, N_TC), jnp.float32),
        compiler_params=pltpu.CompilerParams(
            dimension_semantics=("arbitrary",),
        ),
    )(x2d, W)
    scores_t, idx_t = pl.pallas_call(
        _finish_kernel,
        out_shape=[
            jax.ShapeDtypeStruct((TOPK, N), jnp.float32),
            jax.ShapeDtypeStruct((TOPK, N), jnp.int32),
        ],
    )(cost_sc, cost_tc)
    return (scores_t.T, idx_t.T)


# speedup vs baseline: 2.0014x; 1.0019x over previous
import functools

import jax
import jax.numpy as jnp
from jax import lax
from jax.experimental import pallas as pl
from jax.experimental.pallas import tpu as pltpu
from jax.experimental.pallas import tpu_sc as plsc

SL = 8192
BS = 4
HIDDEN = 1024
EXPERTS = 8
TOPK = 2
N = SL * BS

_TOL = 1e-4
_EPS = 1e-8

N_SC = 8192
N_TC = N - N_SC

NC = 2
NS = 16
L = 16
NW = NC * NS
TPW = N_SC // NW
TB = 16
TG = 4
HC = HIDDEN // L

ROWS = 2048
NT = N_TC // ROWS


def _round_bf16(v):
    u = lax.bitcast_convert_type(v, jnp.uint32)
    lsb = lax.shift_right_logical(u, jnp.uint32(16)) & jnp.uint32(1)
    u = (u + jnp.uint32(0x7FFF) + lsb) & jnp.uint32(0xFFFF0000)
    return lax.bitcast_convert_type(u, jnp.float32)


def _sc_cost_kernel(x_hbm, w_hbm, out_hbm, w_v, x_v, stage_v, o_v):
    cid = lax.axis_index("c")
    sid = lax.axis_index("s")
    wid = sid * NC + cid
    base = wid * TPW
    pltpu.sync_copy(w_hbm, w_v)
    for e in range(EXPERTS):

        def w_round(h, carry, e=e):
            w_v[e, pl.ds(h * L, L)] = _round_bf16(w_v[e, pl.ds(h * L, L)])
            return carry

        lax.fori_loop(0, HC, w_round, 0)
    lane = lax.iota(jnp.int32, L)

    def blk_body(blk, carry):
        pltpu.sync_copy(x_hbm.at[pl.ds(base + blk * TB, TB)], x_v)
        for g in range(TB // TG):

            def h_body(h, accs):
                xcs = [
                    _round_bf16(x_v[g * TG + t, pl.ds(h * L, L)])
                    for t in range(TG)
                ]
                wcs = [w_v[e, pl.ds(h * L, L)] for e in range(EXPERTS)]
                return tuple(
                    accs[t * EXPERTS + e] + xcs[t] * wcs[e]
                    for t in range(TG)
                    for e in range(EXPERTS)
                )

            zero = jnp.zeros((L,), jnp.float32)
            accs = lax.fori_loop(0, HC, h_body, (zero,) * (TG * EXPERTS))
            for t in range(TG):
                r = zero
                for e in range(EXPERTS):
                    r = jnp.where(lane == e, jnp.sum(accs[t * EXPERTS + e]), r)
                stage_v[g * TG + t, :] = r
        for e in range(EXPERTS):
            ge = plsc.load_gather(stage_v, [lane, jnp.full((L,), e, jnp.int32)])
            o_v[e, pl.ds(blk * TB, TB)] = jnp.exp(ge)
        return carry

    lax.fori_loop(0, TPW // TB, blk_body, 0)
    for e in range(EXPERTS):
        pltpu.sync_copy(o_v.at[e], out_hbm.at[e, pl.ds(base, TPW)])


_sc_cost = functools.partial(
    pl.kernel,
    out_type=jax.ShapeDtypeStruct((EXPERTS, N_SC), jnp.float32),
    mesh=plsc.VectorSubcoreMesh(
        core_axis_name="c", subcore_axis_name="s", num_cores=NC, num_subcores=NS
    ),
    scratch_types=[
        pltpu.VMEM((EXPERTS, HIDDEN), jnp.float32),
        pltpu.VMEM((TB, HIDDEN), jnp.float32),
        pltpu.VMEM((TB, L), jnp.float32),
        pltpu.VMEM((EXPERTS, TPW), jnp.float32),
    ],
    compiler_params=pltpu.CompilerParams(needs_layout_passes=False),
    cost_estimate=pl.CostEstimate(
        flops=2 * N_SC * HIDDEN * EXPERTS,
        bytes_accessed=N_SC * HIDDEN * 4 + EXPERTS * N_SC * 4,
        transcendentals=N_SC * EXPERTS,
    ),
)(_sc_cost_kernel)


def _tc_cost_kernel(x_ref, w_ref, out_ref):
    x = x_ref[...]
    w = w_ref[...]
    logits_t = jax.lax.dot_general(
        w, x, (((1,), (1,)), ((), ())), preferred_element_type=jnp.float32
    )
    out_ref[...] = jnp.exp(logits_t)


def _finish_kernel(cost_a_ref, cost_b_ref, scores_ref, idx_ref):
    cost = jnp.concatenate([cost_a_ref[...], cost_b_ref[...]], axis=1)

    def cond_fn(carry):
        return carry[1] > _TOL

    def body_fn(carry):
        d1, _ = carry
        rowsum = jnp.sum(d1 * cost, axis=0, keepdims=True)
        d0 = (1.0 / N) / (rowsum + _EPS)
        colsum = jnp.sum(d0 * cost, axis=1, keepdims=True)
        d1n = (1.0 / EXPERTS) / (colsum + _EPS)
        err = jnp.mean(jnp.abs(d1 - d1n))
        return (d1n, err)

    d1_init = jnp.ones((EXPERTS, 1), jnp.float32)
    d1, _ = lax.while_loop(cond_fn, body_fn, (d1_init, jnp.float32(1e9)))

    s = d1 * cost
    eidx = lax.broadcasted_iota(jnp.int32, (EXPERTS, N), 0)
    m1 = jnp.max(s, axis=0, keepdims=True)
    i1 = jnp.min(jnp.where(s == m1, eidx, EXPERTS), axis=0, keepdims=True)
    masked = jnp.where(eidx == i1, -jnp.inf, s)
    m2 = jnp.max(masked, axis=0, keepdims=True)
    i2 = jnp.min(jnp.where(masked == m2, eidx, EXPERTS), axis=0, keepdims=True)

    denom = jnp.sum(cost, axis=0, keepdims=True)
    c1 = jnp.sum(jnp.where(eidx == i1, cost, 0.0), axis=0, keepdims=True)
    c2 = jnp.sum(jnp.where(eidx == i2, cost, 0.0), axis=0, keepdims=True)
    scores_ref[...] = jnp.concatenate([c1 / denom, c2 / denom], axis=0)
    idx_ref[...] = jnp.concatenate([i1, i2], axis=0)


def kernel(x, W):
    x2d = x.reshape(-1, HIDDEN)
    cost_sc = _sc_cost(x2d, W)
    cost_tc = pl.pallas_call(
        _tc_cost_kernel,
        grid=(NT,),
        in_specs=[
            pl.BlockSpec((ROWS, HIDDEN), lambda i: (i + N_SC // ROWS, 0)),
            pl.BlockSpec((EXPERTS, HIDDEN), lambda i: (0, 0)),
        ],
        out_specs=pl.BlockSpec((EXPERTS, ROWS), lambda i: (0, i)),
        out_shape=jax.ShapeDtypeStruct((EXPERTS, N_TC), jnp.float32),
        compiler_params=pltpu.CompilerParams(
            dimension_semantics=("arbitrary",),
        ),
    )(x2d, W)
    scores_t, idx_t = pl.pallas_call(
        _finish_kernel,
        out_shape=[
            jax.ShapeDtypeStruct((TOPK, N), jnp.float32),
            jax.ShapeDtypeStruct((TOPK, N), jnp.int32),
        ],
    )(cost_sc, cost_tc)
    return (scores_t.T, idx_t.T)


# speedup vs baseline: 4.0711x; 2.0341x over previous
import functools

import jax
import jax.numpy as jnp
from jax import lax
from jax.experimental import pallas as pl
from jax.experimental.pallas import tpu as pltpu
from jax.experimental.pallas import tpu_sc as plsc

SL = 8192
BS = 4
HIDDEN = 1024
EXPERTS = 8
TOPK = 2
N = SL * BS

_TOL = 1e-4
_EPS = 1e-8

N_SC = 8192
N_TC = N - N_SC

NC = 2
NS = 16
L = 16
NW = NC * NS
TPW = N_SC // NW
TB = 16
TG = 4
HC = HIDDEN // L

ROWS = 2048
NT = N_TC // ROWS


def _round_bf16(v):
    u = lax.bitcast_convert_type(v, jnp.uint32)
    lsb = lax.shift_right_logical(u, jnp.uint32(16)) & jnp.uint32(1)
    u = (u + jnp.uint32(0x7FFF) + lsb) & jnp.uint32(0xFFFF0000)
    return lax.bitcast_convert_type(u, jnp.float32)


def _sc_cost_kernel(x_hbm, w_hbm, out_hbm, w_v, x_v, stage_v, o_v):
    cid = lax.axis_index("c")
    sid = lax.axis_index("s")
    wid = sid * NC + cid
    base = wid * TPW
    sl_base = wid * (TPW // BS)
    pltpu.sync_copy(w_hbm, w_v)
    for e in range(EXPERTS):

        def w_round(h, carry, e=e):
            w_v[e, pl.ds(h * L, L)] = _round_bf16(w_v[e, pl.ds(h * L, L)])
            return carry

        lax.fori_loop(0, HC, w_round, 0)
    lane = lax.iota(jnp.int32, L)

    def blk_body(blk, carry):
        pltpu.sync_copy(x_hbm.at[pl.ds(sl_base + blk * (TB // BS), TB // BS)], x_v)
        for g in range(TB // TG):

            def h_body(h, accs):
                xcs = [
                    _round_bf16(
                        x_v[(g * TG + t) // BS, (g * TG + t) % BS, pl.ds(h * L, L)]
                    )
                    for t in range(TG)
                ]
                wcs = [w_v[e, pl.ds(h * L, L)] for e in range(EXPERTS)]
                return tuple(
                    accs[t * EXPERTS + e] + xcs[t] * wcs[e]
                    for t in range(TG)
                    for e in range(EXPERTS)
                )

            zero = jnp.zeros((L,), jnp.float32)
            accs = lax.fori_loop(0, HC, h_body, (zero,) * (TG * EXPERTS))
            for t in range(TG):
                r = zero
                for e in range(EXPERTS):
                    r = jnp.where(lane == e, jnp.sum(accs[t * EXPERTS + e]), r)
                stage_v[g * TG + t, :] = r
        for e in range(EXPERTS):
            ge = plsc.load_gather(stage_v, [lane, jnp.full((L,), e, jnp.int32)])
            o_v[e, pl.ds(blk * TB, TB)] = jnp.exp(ge)
        return carry

    lax.fori_loop(0, TPW // TB, blk_body, 0)
    for e in range(EXPERTS):
        pltpu.sync_copy(o_v.at[e], out_hbm.at[e, pl.ds(base, TPW)])


_sc_cost = functools.partial(
    pl.kernel,
    out_type=jax.ShapeDtypeStruct((EXPERTS, N_SC), jnp.float32),
    mesh=plsc.VectorSubcoreMesh(
        core_axis_name="c", subcore_axis_name="s", num_cores=NC, num_subcores=NS
    ),
    scratch_types=[
        pltpu.VMEM((EXPERTS, HIDDEN), jnp.float32),
        pltpu.VMEM((TB // BS, BS, HIDDEN), jnp.float32),
        pltpu.VMEM((TB, L), jnp.float32),
        pltpu.VMEM((EXPERTS, TPW), jnp.float32),
    ],
    compiler_params=pltpu.CompilerParams(needs_layout_passes=False),
    cost_estimate=pl.CostEstimate(
        flops=2 * N_SC * HIDDEN * EXPERTS,
        bytes_accessed=N_SC * HIDDEN * 4 + EXPERTS * N_SC * 4,
        transcendentals=N_SC * EXPERTS,
    ),
)(_sc_cost_kernel)


def _tc_cost_kernel(x_ref, w_ref, out_ref):
    x = x_ref[...].reshape(ROWS, HIDDEN)
    w = w_ref[...]
    logits_t = jax.lax.dot_general(
        w, x, (((1,), (1,)), ((), ())), preferred_element_type=jnp.float32
    )
    out_ref[...] = jnp.exp(logits_t)


def _finish_kernel(cost_a_ref, cost_b_ref, scores_ref, idx_ref):
    cost = jnp.concatenate([cost_a_ref[...], cost_b_ref[...]], axis=1)

    def cond_fn(carry):
        return carry[1] > _TOL

    def body_fn(carry):
        d1, _ = carry
        rowsum = jnp.sum(d1 * cost, axis=0, keepdims=True)
        d0 = (1.0 / N) / (rowsum + _EPS)
        colsum = jnp.sum(d0 * cost, axis=1, keepdims=True)
        d1n = (1.0 / EXPERTS) / (colsum + _EPS)
        err = jnp.mean(jnp.abs(d1 - d1n))
        return (d1n, err)

    d1_init = jnp.ones((EXPERTS, 1), jnp.float32)
    d1, _ = lax.while_loop(cond_fn, body_fn, (d1_init, jnp.float32(1e9)))

    s = d1 * cost
    eidx = lax.broadcasted_iota(jnp.int32, (EXPERTS, N), 0)
    m1 = jnp.max(s, axis=0, keepdims=True)
    i1 = jnp.min(jnp.where(s == m1, eidx, EXPERTS), axis=0, keepdims=True)
    masked = jnp.where(eidx == i1, -jnp.inf, s)
    m2 = jnp.max(masked, axis=0, keepdims=True)
    i2 = jnp.min(jnp.where(masked == m2, eidx, EXPERTS), axis=0, keepdims=True)

    denom = jnp.sum(cost, axis=0, keepdims=True)
    c1 = jnp.sum(jnp.where(eidx == i1, cost, 0.0), axis=0, keepdims=True)
    c2 = jnp.sum(jnp.where(eidx == i2, cost, 0.0), axis=0, keepdims=True)
    scores_ref[...] = jnp.concatenate([c1 / denom, c2 / denom], axis=0)
    idx_ref[...] = jnp.concatenate([i1, i2], axis=0)


def kernel(x, W):
    cost_sc = _sc_cost(x, W)
    cost_tc = pl.pallas_call(
        _tc_cost_kernel,
        grid=(NT,),
        in_specs=[
            pl.BlockSpec(
                (ROWS // BS, BS, HIDDEN), lambda i: (i + N_SC // ROWS, 0, 0)
            ),
            pl.BlockSpec((EXPERTS, HIDDEN), lambda i: (0, 0)),
        ],
        out_specs=pl.BlockSpec((EXPERTS, ROWS), lambda i: (0, i)),
        out_shape=jax.ShapeDtypeStruct((EXPERTS, N_TC), jnp.float32),
        compiler_params=pltpu.CompilerParams(
            dimension_semantics=("arbitrary",),
            skip_device_barrier=True,
        ),
    )(x, W)
    scores_t, idx_t = pl.pallas_call(
        _finish_kernel,
        out_shape=[
            jax.ShapeDtypeStruct((TOPK, N), jnp.float32),
            jax.ShapeDtypeStruct((TOPK, N), jnp.int32),
        ],
    )(cost_sc, cost_tc)
    return (scores_t.T, idx_t.T)


# speedup vs baseline: 6.1932x; 1.5213x over previous
import functools

import jax
import jax.numpy as jnp
from jax import lax
from jax.experimental import pallas as pl
from jax.experimental.pallas import tpu as pltpu
from jax.experimental.pallas import tpu_sc as plsc

SL = 8192
BS = 4
HIDDEN = 1024
EXPERTS = 8
TOPK = 2
N = SL * BS

_TOL = 1e-4
_EPS = 1e-8

N_SC = 4096
N_TC = N - N_SC

NC = 2
NS = 16
L = 16
NW = NC * NS
TPW = N_SC // NW
TB = 16
TG = 4
HC = HIDDEN // L

ROWS = 2048
NT = N_TC // ROWS


def _round_bf16(v):
    u = lax.bitcast_convert_type(v, jnp.uint32)
    lsb = lax.shift_right_logical(u, jnp.uint32(16)) & jnp.uint32(1)
    u = (u + jnp.uint32(0x7FFF) + lsb) & jnp.uint32(0xFFFF0000)
    return lax.bitcast_convert_type(u, jnp.float32)


def _sc_cost_kernel(x_hbm, w_hbm, out_hbm, w_v, x_v, stage_v, o_v):
    cid = lax.axis_index("c")
    sid = lax.axis_index("s")
    wid = sid * NC + cid
    base = wid * TPW
    sl_base = wid * (TPW // BS)
    pltpu.sync_copy(w_hbm, w_v)
    for e in range(EXPERTS):

        def w_round(h, carry, e=e):
            w_v[e, pl.ds(h * L, L)] = _round_bf16(w_v[e, pl.ds(h * L, L)])
            return carry

        lax.fori_loop(0, HC, w_round, 0)
    lane = lax.iota(jnp.int32, L)

    def blk_body(blk, carry):
        pltpu.sync_copy(x_hbm.at[pl.ds(sl_base + blk * (TB // BS), TB // BS)], x_v)
        for g in range(TB // TG):

            def h_body(h, accs):
                xcs = [
                    _round_bf16(
                        x_v[(g * TG + t) // BS, (g * TG + t) % BS, pl.ds(h * L, L)]
                    )
                    for t in range(TG)
                ]
                wcs = [w_v[e, pl.ds(h * L, L)] for e in range(EXPERTS)]
                return tuple(
                    accs[t * EXPERTS + e] + xcs[t] * wcs[e]
                    for t in range(TG)
                    for e in range(EXPERTS)
                )

            zero = jnp.zeros((L,), jnp.float32)
            accs = lax.fori_loop(0, HC, h_body, (zero,) * (TG * EXPERTS))
            for t in range(TG):
                r = zero
                for e in range(EXPERTS):
                    r = jnp.where(lane == e, jnp.sum(accs[t * EXPERTS + e]), r)
                stage_v[g * TG + t, :] = r
        for e in range(EXPERTS):
            ge = plsc.load_gather(stage_v, [lane, jnp.full((L,), e, jnp.int32)])
            o_v[e, pl.ds(blk * TB, TB)] = jnp.exp(ge)
        return carry

    lax.fori_loop(0, TPW // TB, blk_body, 0)
    for e in range(EXPERTS):
        pltpu.sync_copy(o_v.at[e], out_hbm.at[e, pl.ds(base, TPW)])


_sc_cost = functools.partial(
    pl.kernel,
    out_type=jax.ShapeDtypeStruct((EXPERTS, N_SC), jnp.float32),
    mesh=plsc.VectorSubcoreMesh(
        core_axis_name="c", subcore_axis_name="s", num_cores=NC, num_subcores=NS
    ),
    scratch_types=[
        pltpu.VMEM((EXPERTS, HIDDEN), jnp.float32),
        pltpu.VMEM((TB // BS, BS, HIDDEN), jnp.float32),
        pltpu.VMEM((TB, L), jnp.float32),
        pltpu.VMEM((EXPERTS, TPW), jnp.float32),
    ],
    compiler_params=pltpu.CompilerParams(needs_layout_passes=False),
    cost_estimate=pl.CostEstimate(
        flops=2 * N_SC * HIDDEN * EXPERTS,
        bytes_accessed=N_SC * HIDDEN * 4 + EXPERTS * N_SC * 4,
        transcendentals=N_SC * EXPERTS,
    ),
)(_sc_cost_kernel)


def _tc_cost_kernel(x_ref, w_ref, out_ref):
    x = x_ref[...].reshape(ROWS, HIDDEN)
    w = w_ref[...]
    logits_t = jax.lax.dot_general(
        w, x, (((1,), (1,)), ((), ())), preferred_element_type=jnp.float32
    )
    out_ref[...] = jnp.exp(logits_t)


def _finish_kernel(cost_a_ref, cost_b_ref, scores_ref, idx_ref):
    cost = jnp.concatenate([cost_a_ref[...], cost_b_ref[...]], axis=1)

    def cond_fn(carry):
        return carry[1] > _TOL

    def body_fn(carry):
        d1, _ = carry
        rowsum = jnp.sum(d1 * cost, axis=0, keepdims=True)
        d0 = (1.0 / N) / (rowsum + _EPS)
        colsum = jnp.sum(d0 * cost, axis=1, keepdims=True)
        d1n = (1.0 / EXPERTS) / (colsum + _EPS)
        err = jnp.mean(jnp.abs(d1 - d1n))
        return (d1n, err)

    d1_init = jnp.ones((EXPERTS, 1), jnp.float32)
    d1, _ = lax.while_loop(cond_fn, body_fn, (d1_init, jnp.float32(1e9)))

    s = d1 * cost
    eidx = lax.broadcasted_iota(jnp.int32, (EXPERTS, N), 0)
    m1 = jnp.max(s, axis=0, keepdims=True)
    i1 = jnp.min(jnp.where(s == m1, eidx, EXPERTS), axis=0, keepdims=True)
    masked = jnp.where(eidx == i1, -jnp.inf, s)
    m2 = jnp.max(masked, axis=0, keepdims=True)
    i2 = jnp.min(jnp.where(masked == m2, eidx, EXPERTS), axis=0, keepdims=True)

    denom = jnp.sum(cost, axis=0, keepdims=True)
    c1 = jnp.sum(jnp.where(eidx == i1, cost, 0.0), axis=0, keepdims=True)
    c2 = jnp.sum(jnp.where(eidx == i2, cost, 0.0), axis=0, keepdims=True)
    scores_ref[...] = jnp.concatenate([c1 / denom, c2 / denom], axis=0)
    idx_ref[...] = jnp.concatenate([i1, i2], axis=0)


def kernel(x, W):
    cost_sc = _sc_cost(x, W)
    cost_tc = pl.pallas_call(
        _tc_cost_kernel,
        grid=(NT,),
        in_specs=[
            pl.BlockSpec(
                (ROWS // BS, BS, HIDDEN), lambda i: (i + N_SC // ROWS, 0, 0)
            ),
            pl.BlockSpec((EXPERTS, HIDDEN), lambda i: (0, 0)),
        ],
        out_specs=pl.BlockSpec((EXPERTS, ROWS), lambda i: (0, i)),
        out_shape=jax.ShapeDtypeStruct((EXPERTS, N_TC), jnp.float32),
        compiler_params=pltpu.CompilerParams(
            dimension_semantics=("arbitrary",),
            skip_device_barrier=True,
        ),
    )(x, W)
    scores_t, idx_t = pl.pallas_call(
        _finish_kernel,
        out_shape=[
            jax.ShapeDtypeStruct((TOPK, N), jnp.float32),
            jax.ShapeDtypeStruct((TOPK, N), jnp.int32),
        ],
    )(cost_sc, cost_tc)
    return (scores_t.T, idx_t.T)


# speedup vs baseline: 7.9957x; 1.2911x over previous
import functools

import jax
import jax.numpy as jnp
from jax import lax
from jax.experimental import pallas as pl
from jax.experimental.pallas import tpu as pltpu
from jax.experimental.pallas import tpu_sc as plsc

SL = 8192
BS = 4
HIDDEN = 1024
EXPERTS = 8
TOPK = 2
N = SL * BS

_TOL = 1e-4
_EPS = 1e-8

N_SC = 2048
N_TC = N - N_SC

NC = 2
NS = 16
L = 16
NW = NC * NS
TPW = N_SC // NW
TB = 16
TG = 4
HC = HIDDEN // L

ROWS = 2048
NT = N_TC // ROWS


def _round_bf16(v):
    u = lax.bitcast_convert_type(v, jnp.uint32)
    lsb = lax.shift_right_logical(u, jnp.uint32(16)) & jnp.uint32(1)
    u = (u + jnp.uint32(0x7FFF) + lsb) & jnp.uint32(0xFFFF0000)
    return lax.bitcast_convert_type(u, jnp.float32)


def _sc_cost_kernel(x_hbm, w_hbm, out_hbm, w_v, x_v, stage_v, o_v):
    cid = lax.axis_index("c")
    sid = lax.axis_index("s")
    wid = sid * NC + cid
    base = wid * TPW
    sl_base = wid * (TPW // BS)
    pltpu.sync_copy(w_hbm, w_v)
    for e in range(EXPERTS):

        def w_round(h, carry, e=e):
            w_v[e, pl.ds(h * L, L)] = _round_bf16(w_v[e, pl.ds(h * L, L)])
            return carry

        lax.fori_loop(0, HC, w_round, 0)
    lane = lax.iota(jnp.int32, L)

    def blk_body(blk, carry):
        pltpu.sync_copy(x_hbm.at[pl.ds(sl_base + blk * (TB // BS), TB // BS)], x_v)
        for g in range(TB // TG):

            def h_body(h, accs):
                xcs = [
                    _round_bf16(
                        x_v[(g * TG + t) // BS, (g * TG + t) % BS, pl.ds(h * L, L)]
                    )
                    for t in range(TG)
                ]
                wcs = [w_v[e, pl.ds(h * L, L)] for e in range(EXPERTS)]
                return tuple(
                    accs[t * EXPERTS + e] + xcs[t] * wcs[e]
                    for t in range(TG)
                    for e in range(EXPERTS)
                )

            zero = jnp.zeros((L,), jnp.float32)
            accs = lax.fori_loop(0, HC, h_body, (zero,) * (TG * EXPERTS))
            for t in range(TG):
                r = zero
                for e in range(EXPERTS):
                    r = jnp.where(lane == e, jnp.sum(accs[t * EXPERTS + e]), r)
                stage_v[g * TG + t, :] = r
        for e in range(EXPERTS):
            ge = plsc.load_gather(stage_v, [lane, jnp.full((L,), e, jnp.int32)])
            o_v[e, pl.ds(blk * TB, TB)] = jnp.exp(ge)
        return carry

    lax.fori_loop(0, TPW // TB, blk_body, 0)
    for e in range(EXPERTS):
        pltpu.sync_copy(o_v.at[e], out_hbm.at[e, pl.ds(base, TPW)])


_sc_cost = functools.partial(
    pl.kernel,
    out_type=jax.ShapeDtypeStruct((EXPERTS, N_SC), jnp.float32),
    mesh=plsc.VectorSubcoreMesh(
        core_axis_name="c", subcore_axis_name="s", num_cores=NC, num_subcores=NS
    ),
    scratch_types=[
        pltpu.VMEM((EXPERTS, HIDDEN), jnp.float32),
        pltpu.VMEM((TB // BS, BS, HIDDEN), jnp.float32),
        pltpu.VMEM((TB, L), jnp.float32),
        pltpu.VMEM((EXPERTS, TPW), jnp.float32),
    ],
    compiler_params=pltpu.CompilerParams(needs_layout_passes=False),
    cost_estimate=pl.CostEstimate(
        flops=2 * N_SC * HIDDEN * EXPERTS,
        bytes_accessed=N_SC * HIDDEN * 4 + EXPERTS * N_SC * 4,
        transcendentals=N_SC * EXPERTS,
    ),
)(_sc_cost_kernel)


def _tc_cost_kernel(x_ref, w_ref, out_ref):
    x = x_ref[...].reshape(ROWS, HIDDEN)
    w = w_ref[...]
    logits_t = jax.lax.dot_general(
        w, x, (((1,), (1,)), ((), ())), preferred_element_type=jnp.float32
    )
    out_ref[...] = jnp.exp(logits_t)


def _finish_kernel(cost_a_ref, cost_b_ref, scores_ref, idx_ref):
    cost = jnp.concatenate([cost_a_ref[...], cost_b_ref[...]], axis=1)

    def cond_fn(carry):
        return carry[1] > _TOL

    def body_fn(carry):
        d1, _ = carry
        rowsum = jnp.sum(d1 * cost, axis=0, keepdims=True)
        d0 = (1.0 / N) / (rowsum + _EPS)
        colsum = jnp.sum(d0 * cost, axis=1, keepdims=True)
        d1n = (1.0 / EXPERTS) / (colsum + _EPS)
        err = jnp.mean(jnp.abs(d1 - d1n))
        return (d1n, err)

    d1_init = jnp.ones((EXPERTS, 1), jnp.float32)
    d1, _ = lax.while_loop(cond_fn, body_fn, (d1_init, jnp.float32(1e9)))

    s = d1 * cost
    eidx = lax.broadcasted_iota(jnp.int32, (EXPERTS, N), 0)
    m1 = jnp.max(s, axis=0, keepdims=True)
    i1 = jnp.min(jnp.where(s == m1, eidx, EXPERTS), axis=0, keepdims=True)
    masked = jnp.where(eidx == i1, -jnp.inf, s)
    m2 = jnp.max(masked, axis=0, keepdims=True)
    i2 = jnp.min(jnp.where(masked == m2, eidx, EXPERTS), axis=0, keepdims=True)

    denom = jnp.sum(cost, axis=0, keepdims=True)
    c1 = jnp.sum(jnp.where(eidx == i1, cost, 0.0), axis=0, keepdims=True)
    c2 = jnp.sum(jnp.where(eidx == i2, cost, 0.0), axis=0, keepdims=True)
    scores_ref[...] = jnp.concatenate([c1 / denom, c2 / denom], axis=0)
    idx_ref[...] = jnp.concatenate([i1, i2], axis=0)


def kernel(x, W):
    cost_sc = _sc_cost(x, W)
    cost_tc = pl.pallas_call(
        _tc_cost_kernel,
        grid=(NT,),
        in_specs=[
            pl.BlockSpec(
                (ROWS // BS, BS, HIDDEN), lambda i: (i + N_SC // ROWS, 0, 0)
            ),
            pl.BlockSpec((EXPERTS, HIDDEN), lambda i: (0, 0)),
        ],
        out_specs=pl.BlockSpec((EXPERTS, ROWS), lambda i: (0, i)),
        out_shape=jax.ShapeDtypeStruct((EXPERTS, N_TC), jnp.float32),
        compiler_params=pltpu.CompilerParams(
            dimension_semantics=("arbitrary",),
            skip_device_barrier=True,
        ),
    )(x, W)
    scores_t, idx_t = pl.pallas_call(
        _finish_kernel,
        out_shape=[
            jax.ShapeDtypeStruct((TOPK, N), jnp.float32),
            jax.ShapeDtypeStruct((TOPK, N), jnp.int32),
        ],
    )(cost_sc, cost_tc)
    return (scores_t.T, idx_t.T)
